# baseline (speedup 1.0000x reference)
import jax
import jax.numpy as jnp
from jax.experimental import pallas as pl


_COLS = 6272


def _mul_kernel(um_ref, x_ref, o_ref):
    m = um_ref[0, :]
    o_ref[...] = x_ref[...] * m[None, :, None]


def kernel(input, unit_mapping):
    B, C, H, W = input.shape
    HW = H * W
    x = input.reshape(B, C, HW)
    grid = (B, HW // _COLS)
    out = pl.pallas_call(
        _mul_kernel,
        grid=grid,
        in_specs=[
            pl.BlockSpec((unit_mapping.shape[0], C), lambda b, s: (0, 0)),
            pl.BlockSpec((1, C, _COLS), lambda b, s: (b, 0, s)),
        ],
        out_specs=pl.BlockSpec((1, C, _COLS), lambda b, s: (b, 0, s)),
        out_shape=jax.ShapeDtypeStruct((B, C, HW), input.dtype),
    )(unit_mapping, x)
    return out.reshape(B, C, H, W)


# speedup vs baseline: 1.0096x; 1.0096x over previous
import jax
import jax.numpy as jnp
from jax.experimental import pallas as pl


_ROWS = 16


def _mul_kernel(um_ref, x_ref, o_ref):
    m = um_ref[:, 0]
    o_ref[...] = x_ref[...] * m[:, None]


def kernel(input, unit_mapping):
    B, C, H, W = input.shape
    HW = H * W
    x = input.reshape(B * C, HW)
    um_t = unit_mapping.T
    nblk = C // _ROWS
    grid = (B * C // _ROWS,)
    out = pl.pallas_call(
        _mul_kernel,
        grid=grid,
        in_specs=[
            pl.BlockSpec((_ROWS, um_t.shape[1]), lambda g: (g % nblk, 0)),
            pl.BlockSpec((_ROWS, HW), lambda g: (g, 0)),
        ],
        out_specs=pl.BlockSpec((_ROWS, HW), lambda g: (g, 0)),
        out_shape=jax.ShapeDtypeStruct((B * C, HW), input.dtype),
    )(um_t, x)
    return out.reshape(B, C, H, W)


# speedup vs baseline: 3.8323x; 3.7958x over previous
import jax
import jax.numpy as jnp
from jax.experimental import pallas as pl


_CB = 16


def _mul_kernel(um_ref, x_ref, o_ref):
    m = um_ref[:, 0]
    o_ref[...] = x_ref[...] * m[None, :, None, None]


def kernel(input, unit_mapping):
    B, C, H, W = input.shape
    um_t = unit_mapping.T
    grid = (B, C // _CB)
    out = pl.pallas_call(
        _mul_kernel,
        grid=grid,
        in_specs=[
            pl.BlockSpec((_CB, um_t.shape[1]), lambda b, c: (c, 0)),
            pl.BlockSpec((1, _CB, H, W), lambda b, c: (b, c, 0, 0)),
        ],
        out_specs=pl.BlockSpec((1, _CB, H, W), lambda b, c: (b, c, 0, 0)),
        out_shape=jax.ShapeDtypeStruct((B, C, H, W), input.dtype),
    )(um_t, input)
    return out


# speedup vs baseline: 3.9012x; 1.0180x over previous
import jax
import jax.numpy as jnp
from jax.experimental import pallas as pl
from jax.experimental.pallas import tpu as pltpu


_CB = 32


def _mul_kernel(um_ref, x_ref, o_ref):
    m = um_ref[:, 0]
    o_ref[...] = x_ref[...] * m[None, :, None, None]


def kernel(input, unit_mapping):
    B, C, H, W = input.shape
    um_t = unit_mapping.T
    grid = (B, C // _CB)
    out = pl.pallas_call(
        _mul_kernel,
        grid=grid,
        in_specs=[
            pl.BlockSpec((_CB, um_t.shape[1]), lambda b, c: (c, 0)),
            pl.BlockSpec((1, _CB, H, W), lambda b, c: (b, c, 0, 0)),
        ],
        out_specs=pl.BlockSpec((1, _CB, H, W), lambda b, c: (b, c, 0, 0)),
        out_shape=jax.ShapeDtypeStruct((B, C, H, W), input.dtype),
        compiler_params=pltpu.CompilerParams(
            dimension_semantics=("parallel", "parallel")),
    )(um_t, input)
    return out


# speedup vs baseline: 3.9128x; 1.0030x over previous
import jax
import jax.numpy as jnp
from jax.experimental import pallas as pl
from jax.experimental.pallas import tpu as pltpu


_CB = 48


def _mul_kernel(um_ref, x_ref, o_ref):
    m = um_ref[:, 0]
    o_ref[...] = x_ref[...] * m[None, :, None, None]


def kernel(input, unit_mapping):
    B, C, H, W = input.shape
    um_t = unit_mapping.T
    grid = (B, C // _CB)
    out = pl.pallas_call(
        _mul_kernel,
        grid=grid,
        in_specs=[
            pl.BlockSpec((_CB, um_t.shape[1]), lambda b, c: (c, 0)),
            pl.BlockSpec((1, _CB, H, W), lambda b, c: (b, c, 0, 0)),
        ],
        out_specs=pl.BlockSpec((1, _CB, H, W), lambda b, c: (b, c, 0, 0)),
        out_shape=jax.ShapeDtypeStruct((B, C, H, W), input.dtype),
        compiler_params=pltpu.CompilerParams(
            dimension_semantics=("parallel", "parallel")),
    )(um_t, input)
    return out


# speedup vs baseline: 3.9591x; 1.0118x over previous
import jax
import jax.numpy as jnp
from jax.experimental import pallas as pl
from jax.experimental.pallas import tpu as pltpu


_CB = 48


def _mul_kernel(um_ref, x_ref, o_ref):
    c0 = pl.program_id(1) * _CB
    for i in range(_CB):
        s = um_ref[0, c0 + i]
        o_ref[0, i] = x_ref[0, i] * s


def kernel(input, unit_mapping):
    B, C, H, W = input.shape
    grid = (B, C // _CB)
    out = pl.pallas_call(
        _mul_kernel,
        grid=grid,
        in_specs=[
            pl.BlockSpec(memory_space=pltpu.SMEM),
            pl.BlockSpec((1, _CB, H, W), lambda b, c: (b, c, 0, 0)),
        ],
        out_specs=pl.BlockSpec((1, _CB, H, W), lambda b, c: (b, c, 0, 0)),
        out_shape=jax.ShapeDtypeStruct((B, C, H, W), input.dtype),
        compiler_params=pltpu.CompilerParams(
            dimension_semantics=("parallel", "parallel")),
    )(unit_mapping, input)
    return out
